# Initial kernel scaffold; baseline (speedup 1.0000x reference)
import jax
import jax.numpy as jnp
from jax.experimental import pallas as pl


def kernel(x, edge_index, edge_attr, W1, b1, W2, b2, W3, b3, W4, b4):
    raise NotImplementedError("write your pallas kernel here")


# speedup vs baseline: 2.2124x; 2.2124x over previous
import jax
import jax.numpy as jnp
from jax.experimental import pallas as pl
from jax.experimental.pallas import tpu as pltpu

_N = 10000
_BLOCK_N = 1000


def _mlp_block(x_ref, w1_ref, b1_ref, w2_ref, b2_ref, w3_ref, b3_ref,
               w4_ref, b4_ref, out_ref):
    h = jnp.dot(x_ref[:], w1_ref[:], preferred_element_type=jnp.float32)
    h += b1_ref[:]
    h = jnp.where(h > 0, h, (jnp.exp(h) - 1.0))
    h = jnp.dot(h, w2_ref[:], preferred_element_type=jnp.float32)
    h += b2_ref[:]
    h = jnp.where(h > 0, h, (jnp.exp(h) - 1.0))
    h = jnp.dot(h, w3_ref[:], preferred_element_type=jnp.float32)
    h += b3_ref[:]
    h = jnp.where(h > 0, h, (jnp.exp(h) - 1.0))
    h = jnp.dot(h, w4_ref[:], preferred_element_type=jnp.float32)
    h += b4_ref[:]
    out_ref[:] = jnp.where(h > 0, h, 256.0 * (jnp.exp(h) - 1.0))


def kernel(x, edge_index, edge_attr, W1, b1, W2, b2, W3, b3, W4, b4):
    del edge_index, edge_attr
    n, d_in = x.shape
    d_out = W4.shape[1]
    block_n = _BLOCK_N if n % _BLOCK_N == 0 else n
    grid = (n // block_n,)

    def _rows(i):
        return (i, 0)

    def _whole(i):
        return (0, 0)

    return pl.pallas_call(
        _mlp_block,
        grid=grid,
        in_specs=[
            pl.BlockSpec((block_n, d_in), _rows),
            pl.BlockSpec(W1.shape, _whole),
            pl.BlockSpec((1, b1.shape[0]), _whole),
            pl.BlockSpec(W2.shape, _whole),
            pl.BlockSpec((1, b2.shape[0]), _whole),
            pl.BlockSpec(W3.shape, _whole),
            pl.BlockSpec((1, b3.shape[0]), _whole),
            pl.BlockSpec(W4.shape, _whole),
            pl.BlockSpec((1, b4.shape[0]), _whole),
        ],
        out_specs=pl.BlockSpec((block_n, d_out), _rows),
        out_shape=jax.ShapeDtypeStruct((n, d_out), jnp.float32),
        compiler_params=pltpu.CompilerParams(
            dimension_semantics=("arbitrary",),
        ),
    )(x, W1, b1.reshape(1, -1), W2, b2.reshape(1, -1),
      W3, b3.reshape(1, -1), W4, b4.reshape(1, -1))
